# baseline (speedup 1.0000x reference)
import functools

import jax
import jax.numpy as jnp
from jax import lax
from jax.experimental import pallas as pl
from jax.experimental.pallas import tpu as pltpu
from jax.experimental.pallas import tpu_sc as plsc

BATCH = 4
SEQ = 8192
D_MODEL = 1024

_NUM_CORES = 2
_NUM_SUBCORES = 16
_NW = _NUM_CORES * _NUM_SUBCORES
_S_PER_W = SEQ // _NW
_G = 8
_NG = _S_PER_W // _G
_LANES = 16
_CHUNKS = _G * D_MODEL // _LANES


def _body(x_hbm, idx_hbm, pe_hbm, out_hbm, idx_v, rows_v, xb_v, sem):
    wid = lax.axis_index("s") * _NUM_CORES + lax.axis_index("c")
    base = wid * _S_PER_W
    pltpu.sync_copy(idx_hbm.at[pl.ds(base, _S_PER_W)], idx_v)

    def group(g, _):
        row0 = base + g * _G
        pltpu.async_copy(
            pe_hbm.at[idx_v.at[pl.ds(g * _G, _G)]], rows_v, sem
        ).wait()

        def batch_body(b, __):
            pltpu.sync_copy(x_hbm.at[b, pl.ds(row0, _G)], xb_v)

            def chunk(i, ___):
                r = i // (D_MODEL // _LANES)
                c = (i % (D_MODEL // _LANES)) * _LANES
                xb_v[r, pl.ds(c, _LANES)] = (
                    xb_v[r, pl.ds(c, _LANES)] + rows_v[r, pl.ds(c, _LANES)]
                )
                return ___

            lax.fori_loop(0, _CHUNKS, chunk, 0, unroll=8)
            pltpu.sync_copy(xb_v, out_hbm.at[b, pl.ds(row0, _G)])
            return __

        lax.fori_loop(0, BATCH, batch_body, 0)
        return _

    lax.fori_loop(0, _NG, group, 0)


@jax.jit
def _pe_add(x, idx, pe):
    mesh = plsc.VectorSubcoreMesh(core_axis_name="c", subcore_axis_name="s")
    return pl.kernel(
        _body,
        mesh=mesh,
        out_type=jax.ShapeDtypeStruct((BATCH, SEQ, D_MODEL), jnp.float32),
        scratch_types=[
            pltpu.VMEM((_S_PER_W,), jnp.int32),
            pltpu.VMEM((_G, D_MODEL), jnp.float32),
            pltpu.VMEM((_G, D_MODEL), jnp.float32),
            pltpu.SemaphoreType.DMA,
        ],
    )(x, idx, pe)


def kernel(x, x_node_inds, pe, device=0):
    idx = x_node_inds.astype(jnp.int32)
    return _pe_add(x, idx, pe)


# speedup vs baseline: 2.8120x; 2.8120x over previous
import jax
import jax.numpy as jnp
from jax import lax
from jax.experimental import pallas as pl
from jax.experimental.pallas import tpu as pltpu
from jax.experimental.pallas import tpu_sc as plsc

BATCH = 4
SEQ = 8192
D_MODEL = 1024

_NUM_CORES = 2
_NUM_SUBCORES = 16
_NW = _NUM_CORES * _NUM_SUBCORES
_S_PER_W = SEQ // _NW
_G = 8
_NG = _S_PER_W // _G
_NJ = _NG // 2
_LANES = 16
_CPR = D_MODEL // _LANES
_POS = _G * _CPR


def _body(x_hbm, idx_hbm, pe_hbm, out_hbm,
          idx_v, rows_v, xb_v, gs0, gs1, xi0, xi1, xo0, xo1):
    wid = lax.axis_index("s") * _NUM_CORES + lax.axis_index("c")
    base = wid * _S_PER_W
    gsem = (gs0, gs1)
    xisem = (xi0, xi1)
    xosem = (xo0, xo1)

    pltpu.sync_copy(idx_hbm.at[pl.ds(base, _S_PER_W)], idx_v)

    def gather(p, g):
        return pltpu.make_async_copy(
            pe_hbm.at[idx_v.at[pl.ds(g * _G, _G)]], rows_v.at[p], gsem[p]
        )

    def xin(p, b, g):
        return pltpu.make_async_copy(
            x_hbm.at[b, pl.ds(base + g * _G, _G)], xb_v.at[p, b], xisem[p]
        )

    def xout(p, b, g):
        return pltpu.make_async_copy(
            xb_v.at[p, b], out_hbm.at[b, pl.ds(base + g * _G, _G)], xosem[p]
        )

    def compute(p):
        def chunk(i, carry):
            r = i // _CPR
            c = (i % _CPR) * _LANES
            pe_chunk = rows_v[p, r, pl.ds(c, _LANES)]
            for b in range(BATCH):
                xb_v[p, b, r, pl.ds(c, _LANES)] = (
                    xb_v[p, b, r, pl.ds(c, _LANES)] + pe_chunk
                )
            return carry

        lax.fori_loop(0, _POS, chunk, 0, unroll=4)

    def section(p, g, prefetch):
        gather(p, g).wait()
        for b in range(BATCH):
            xin(p, b, g).wait()
        compute(p)
        for b in range(BATCH):
            xout(p, b, g).start()
        if prefetch:
            gather(p, g + 2).start()
            for b in range(BATCH):
                xout(p, b, g).wait()
                xin(p, b, g + 2).start()

    gather(0, 0).start()
    gather(1, 1).start()
    for b in range(BATCH):
        xin(0, b, 0).start()
        xin(1, b, 1).start()

    def jbody(j, carry):
        g0 = j * 2
        section(0, g0, prefetch=True)
        section(1, g0 + 1, prefetch=True)
        return carry

    lax.fori_loop(0, _NJ - 1, jbody, 0)
    g0 = (_NJ - 1) * 2
    section(0, g0, prefetch=False)
    section(1, g0 + 1, prefetch=False)


@jax.jit
def _pe_add(x, idx, pe):
    mesh = plsc.VectorSubcoreMesh(core_axis_name="c", subcore_axis_name="s")
    return pl.kernel(
        _body,
        mesh=mesh,
        out_type=jax.ShapeDtypeStruct((BATCH, SEQ, D_MODEL), jnp.float32),
        scratch_types=[
            pltpu.VMEM((_S_PER_W,), jnp.int32),
            pltpu.VMEM((2, _G, D_MODEL), jnp.float32),
            pltpu.VMEM((2, BATCH, _G, D_MODEL), jnp.float32),
            pltpu.SemaphoreType.DMA,
            pltpu.SemaphoreType.DMA,
            pltpu.SemaphoreType.DMA,
            pltpu.SemaphoreType.DMA,
            pltpu.SemaphoreType.DMA,
            pltpu.SemaphoreType.DMA,
        ],
    )(x, idx, pe)


def kernel(x, x_node_inds, pe, device=0):
    idx = x_node_inds.astype(jnp.int32)
    return _pe_add(x, idx, pe)
